# Initial kernel scaffold; baseline (speedup 1.0000x reference)
import functools

import jax
import jax.numpy as jnp
from jax import lax
from jax.experimental import pallas as pl
from jax.experimental.pallas import tpu as pltpu
from jax.experimental.pallas import tpu_sc as plsc

N = 10000
E = 320000
D = 128

NC = 2
NS = 16
NW = NC * NS

K = 128
CHUNKS = E // K
FULL_ROUNDS = CHUNKS // NW
REM = CHUNKS - FULL_ROUNDS * NW
ROWS_PER_TILE = N // NS


def _sc_partial(edge_attr, dst2, zeros):
    mesh = plsc.VectorSubcoreMesh(core_axis_name="c", subcore_axis_name="s")

    @functools.partial(
        pl.kernel,
        out_type=jax.ShapeDtypeStruct((NC, N, D), jnp.float32),
        mesh=mesh,
        scratch_types=[
            pltpu.VMEM((1, K), jnp.int32),
            pltpu.VMEM((K, D), jnp.float32),
            pltpu.VMEM_SHARED((N, D), jnp.float32),
        ],
    )
    def k(edge_hbm, dst_hbm, zeros_hbm, part_hbm, idx_v, rows_v, acc):
        c = lax.axis_index("c")
        s = lax.axis_index("s")
        wid = s * NC + c

        pltpu.sync_copy(
            zeros_hbm.at[pl.ds(s * ROWS_PER_TILE, ROWS_PER_TILE)],
            acc.at[pl.ds(s * ROWS_PER_TILE, ROWS_PER_TILE)],
        )
        plsc.subcore_barrier()

        def do_chunk(cid):
            base = cid * K
            pltpu.sync_copy(dst_hbm.at[pl.ds(cid, 1)], idx_v)
            pltpu.sync_copy(edge_hbm.at[pl.ds(base, K), pl.ds(1, D)], rows_v)
            pltpu.sync_copy(rows_v, acc.at[idx_v.at[0]], add=True)

        def body(j, carry):
            do_chunk(j * NW + wid)
            return carry

        lax.fori_loop(0, FULL_ROUNDS, body, 0)

        @pl.when(wid < REM)
        def _():
            do_chunk(FULL_ROUNDS * NW + wid)

        plsc.subcore_barrier()

        pltpu.sync_copy(
            acc.at[pl.ds(s * ROWS_PER_TILE, ROWS_PER_TILE)],
            part_hbm.at[c, pl.ds(s * ROWS_PER_TILE, ROWS_PER_TILE)],
        )

    return k(edge_attr, dst2, zeros)


def _combine(vertex_attr, partial):
    def body(v_ref, p_ref, o_ref):
        o_ref[:, :D] = v_ref[...]
        o_ref[:, D:] = p_ref[0] + p_ref[1]

    return pl.pallas_call(
        body,
        grid=(20,),
        in_specs=[
            pl.BlockSpec((500, D), lambda i: (i, 0)),
            pl.BlockSpec((NC, 500, D), lambda i: (0, i, 0)),
        ],
        out_specs=pl.BlockSpec((500, 2 * D), lambda i: (i, 0)),
        out_shape=jax.ShapeDtypeStruct((N, 2 * D), jnp.float32),
    )(vertex_attr, partial)


def kernel(vertex_attr, edgeij_pair, edge_attr, g, batch):
    dst2 = edgeij_pair[1].reshape(CHUNKS, K)
    zeros = jnp.zeros((N, D), dtype=jnp.float32)
    partial = _sc_partial(edge_attr, dst2, zeros)
    return _combine(vertex_attr, partial)


# speedup vs baseline: 2.5009x; 2.5009x over previous
import functools

import jax
import jax.numpy as jnp
from jax import lax
from jax.experimental import pallas as pl
from jax.experimental.pallas import tpu as pltpu
from jax.experimental.pallas import tpu_sc as plsc

N = 10000
E = 320000
D = 128

NC = 2
NS = 16
NW = NC * NS
L = 16

K = 128
CHUNKS = E // K
SUPERS = E // (8 * K)
FULL_ROUNDS = SUPERS // NW
SUP_REM = SUPERS - FULL_ROUNDS * NW
REM_CHUNKS = CHUNKS - SUPERS * 8
REM_BASE = SUPERS * 8

N_PAD = 10240
SLAB = N_PAD // NS


def _sc_partial(edge_attr, dst2, lastc2, zeros):
    mesh = plsc.VectorSubcoreMesh(core_axis_name="c", subcore_axis_name="s")

    @functools.partial(
        pl.kernel,
        out_type=jax.ShapeDtypeStruct((NC, N_PAD, D), jnp.float32),
        mesh=mesh,
        scratch_types=[
            pltpu.VMEM((8, K), jnp.int32),
            pltpu.VMEM((8, K), jnp.float32),
            pltpu.VMEM((K, D), jnp.float32),
            pltpu.VMEM_SHARED((N_PAD, D), jnp.float32),
        ],
        compiler_params=pltpu.CompilerParams(needs_layout_passes=False),
    )
    def k(edge_hbm, dst_hbm, lastc_hbm, zeros_hbm, part_hbm,
          idx_v, last_v, rows_v, acc):
        c = lax.axis_index("c")
        s = lax.axis_index("s")
        wid = s * NC + c
        lane = lax.iota(jnp.int32, L)
        zero16 = jnp.zeros((L,), jnp.int32)

        pltpu.sync_copy(
            zeros_hbm.at[pl.ds(s * SLAB, SLAB)],
            acc.at[pl.ds(s * SLAB, SLAB)],
        )
        plsc.subcore_barrier()

        def do_chunk(t, base):
            pltpu.sync_copy(edge_hbm.at[pl.ds(base, K), pl.ds(0, D)], rows_v)
            for i in range(K // L):
                vals = last_v[t, pl.ds(i * L, L)]
                plsc.store_scatter(rows_v, [lane + i * L, zero16], vals)
            pltpu.sync_copy(rows_v, acc.at[idx_v.at[t]], add=True)

        def do_super(sid):
            pltpu.sync_copy(dst_hbm.at[pl.ds(sid * 8, 8)], idx_v)
            pltpu.sync_copy(lastc_hbm.at[pl.ds(sid * 8, 8)], last_v)
            for t in range(8):
                do_chunk(t, (sid * 8 + t) * K)

        def body(j, carry):
            do_super(j * NW + wid)
            return carry

        lax.fori_loop(0, FULL_ROUNDS, body, 0)

        @pl.when(wid < SUP_REM)
        def _():
            do_super(FULL_ROUNDS * NW + wid)

        @pl.when(jnp.logical_and(wid >= SUP_REM, wid < SUP_REM + REM_CHUNKS))
        def _():
            t = wid - SUP_REM
            pltpu.sync_copy(dst_hbm.at[pl.ds(REM_BASE, 8)], idx_v)
            pltpu.sync_copy(lastc_hbm.at[pl.ds(REM_BASE, 8)], last_v)
            for u in range(REM_CHUNKS):
                @pl.when(t == u)
                def _():
                    do_chunk(u, (REM_BASE + u) * K)

        plsc.subcore_barrier()

        pltpu.sync_copy(
            acc.at[pl.ds(s * SLAB, SLAB)],
            part_hbm.at[c, pl.ds(s * SLAB, SLAB)],
        )

    return k(edge_attr, dst2, lastc2, zeros)


def _combine(vertex_attr, partial):
    def body(v_ref, p_ref, o_ref):
        p = p_ref[0] + p_ref[1]
        o_ref[:, :D] = v_ref[...]
        o_ref[:, D:] = jnp.concatenate([p[:, 1:], p[:, :1]], axis=1)

    return pl.pallas_call(
        body,
        grid=(10,),
        in_specs=[
            pl.BlockSpec((1000, D), lambda i: (i, 0)),
            pl.BlockSpec((NC, 1000, D), lambda i: (0, i, 0)),
        ],
        out_specs=pl.BlockSpec((1000, 2 * D), lambda i: (i, 0)),
        out_shape=jax.ShapeDtypeStruct((N, 2 * D), jnp.float32),
    )(vertex_attr, partial)


def kernel(vertex_attr, edgeij_pair, edge_attr, g, batch):
    pad = (-CHUNKS) % 8
    dst2 = jnp.pad(edgeij_pair[1].reshape(CHUNKS, K), ((0, pad), (0, 0)))
    lastc2 = jnp.pad(edge_attr[:, D].reshape(CHUNKS, K), ((0, pad), (0, 0)))
    zeros = jnp.zeros((N_PAD, D), dtype=jnp.float32)
    partial = _sc_partial(edge_attr, dst2, lastc2, zeros)
    return _combine(vertex_attr, partial)


# speedup vs baseline: 2.9148x; 1.1655x over previous
import functools

import jax
import jax.numpy as jnp
from jax import lax
from jax.experimental import pallas as pl
from jax.experimental.pallas import tpu as pltpu
from jax.experimental.pallas import tpu_sc as plsc

N = 10000
E = 320000
D = 128

NC = 2
NS = 16
NW = NC * NS
L = 16

K = 80
CHUNKS = E // K
M_MAX = CHUNKS // NW
NBUF = 4
P = 2

N_PAD = 10240
SLAB = N_PAD // NS


def _sc_partial(edge_attr, il3, zeros):
    mesh = plsc.VectorSubcoreMesh(core_axis_name="c", subcore_axis_name="s")

    @functools.partial(
        pl.kernel,
        out_type=jax.ShapeDtypeStruct((NC, N_PAD, D), jnp.float32),
        mesh=mesh,
        scratch_types=[
            pltpu.VMEM((NBUF, 2, K), jnp.int32),
            pltpu.VMEM((NBUF, K, D), jnp.float32),
            pltpu.VMEM_SHARED((N_PAD, D), jnp.float32),
            pltpu.SemaphoreType.DMA((NBUF,)),
            pltpu.SemaphoreType.DMA((NBUF,)),
            pltpu.SemaphoreType.DMA((NBUF,)),
        ],
        compiler_params=pltpu.CompilerParams(needs_layout_passes=False),
    )
    def k(edge_hbm, il_hbm, zeros_hbm, part_hbm,
          il_v, rows_v, acc, msem, rsem, ssem):
        c = lax.axis_index("c")
        s = lax.axis_index("s")
        wid = s * NC + c
        lane = lax.iota(jnp.int32, L)
        zero16 = jnp.zeros((L,), jnp.int32)

        pltpu.sync_copy(
            zeros_hbm.at[pl.ds(s * SLAB, SLAB)],
            acc.at[pl.ds(s * SLAB, SLAB)],
        )
        plsc.subcore_barrier()

        def issue_gathers(mm, b):
            cid = mm * NW + wid
            pltpu.async_copy(il_hbm.at[cid], il_v.at[b], msem.at[b])
            pltpu.async_copy(
                edge_hbm.at[pl.ds(cid * K, K), pl.ds(0, D)],
                rows_v.at[b], rsem.at[b])

        def wait_meta(b):
            pltpu.make_async_copy(il_hbm.at[0], il_v.at[b], msem.at[b]).wait()

        def wait_rows(b):
            pltpu.make_async_copy(
                edge_hbm.at[pl.ds(0, K), pl.ds(0, D)],
                rows_v.at[b], rsem.at[b]).wait()

        def wait_scat(b):
            pltpu.make_async_copy(
                rows_v.at[b], acc.at[il_v.at[b, 0]], ssem.at[b]).wait()

        def consume(b):
            wait_meta(b)
            wait_rows(b)
            for i in range(K // L):
                vals = plsc.bitcast(
                    il_v[b, 1, pl.ds(i * L, L)], jnp.float32)
                plsc.store_scatter(
                    rows_v.at[b], [lane + i * L, zero16], vals)
            pltpu.async_copy(
                rows_v.at[b], acc.at[il_v.at[b, 0]], ssem.at[b], add=True)

        for p in range(P):
            issue_gathers(p, p)

        ROUNDS = M_MAX // NBUF

        def body(r, carry):
            for b in range(NBUF):
                mm = r * NBUF + b
                consume(b)
                kk = mm + P
                kb = (b + P) % NBUF

                @pl.when(kk < M_MAX)
                def _():
                    @pl.when(kk >= NBUF)
                    def _():
                        wait_scat(kb)
                    issue_gathers(kk, kb)
            return carry

        lax.fori_loop(0, ROUNDS, body, 0)

        for mm in range(ROUNDS * NBUF, M_MAX):
            consume(mm % NBUF)

        for mm in range(M_MAX - NBUF, M_MAX):
            wait_scat(mm % NBUF)

        plsc.subcore_barrier()

        pltpu.sync_copy(
            acc.at[pl.ds(s * SLAB, SLAB)],
            part_hbm.at[c, pl.ds(s * SLAB, SLAB)],
        )

    return k(edge_attr, il3, zeros)


def _combine(vertex_attr, partial):
    def body(v_ref, p_ref, o_ref):
        p = p_ref[0] + p_ref[1]
        o_ref[:, :D] = v_ref[...]
        o_ref[:, D:] = jnp.concatenate([p[:, 1:], p[:, :1]], axis=1)

    return pl.pallas_call(
        body,
        grid=(10,),
        in_specs=[
            pl.BlockSpec((1000, D), lambda i: (i, 0)),
            pl.BlockSpec((NC, 1000, D), lambda i: (0, i, 0)),
        ],
        out_specs=pl.BlockSpec((1000, 2 * D), lambda i: (i, 0)),
        out_shape=jax.ShapeDtypeStruct((N, 2 * D), jnp.float32),
    )(vertex_attr, partial)


def kernel(vertex_attr, edgeij_pair, edge_attr, g, batch):
    dst2 = edgeij_pair[1].reshape(CHUNKS, 1, K)
    last2 = lax.bitcast_convert_type(
        edge_attr[:, D].reshape(CHUNKS, 1, K), jnp.int32)
    il3 = jnp.concatenate([dst2, last2], axis=1)
    zeros = jnp.zeros((N_PAD, D), dtype=jnp.float32)
    partial = _sc_partial(edge_attr, il3, zeros)
    return _combine(vertex_attr, partial)
